# baseline (speedup 1.0000x reference)
import functools

import jax
import jax.numpy as jnp
from jax import lax
from jax.experimental import pallas as pl
from jax.experimental.pallas import tpu as pltpu

_CENTER_VAR = 0.1
_SIZE_VAR = 0.2
_IOU_T = 0.45
_SCORE_T = 0.01
_TOPK = 100
_NEG = -1e30


def _prologue(lt_ref, bb_ref, pr_ref, probs_ref, box_ref, *, C):
    x = lt_ref[...]
    m = jnp.max(x, axis=0, keepdims=True)
    e = jnp.exp(x - m)
    d = jnp.sum(e, axis=0, keepdims=True)
    probs_ref[...] = e / d

    loc = bb_ref[...]
    p = pr_ref[...]
    cx = loc[0] * _CENTER_VAR * p[2] + p[0]
    cy = loc[1] * _CENTER_VAR * p[3] + p[1]
    w = jnp.exp(loc[2] * _SIZE_VAR) * p[2]
    h = jnp.exp(loc[3] * _SIZE_VAR) * p[3]
    box_ref[...] = jnp.stack(
        [cx - w * 0.5, cy - h * 0.5, cx + w * 0.5, cy + h * 0.5], axis=0
    )


def _perclass(probs_ref, boxT_ref, out_ref, s_ref, cm_ref, iou_ref, *, N, R, K):
    CH = s_ref.shape[0] // 8
    rows_i = lax.broadcasted_iota(jnp.int32, (CH * 8, 128), 0)
    lanes_i = lax.broadcasted_iota(jnp.int32, (CH * 8, 128), 1)
    flat = rows_i * 128 + lanes_i

    p = probs_ref[0]
    if CH * 8 > R:
        p = jnp.concatenate(
            [p, jnp.full((CH * 8 - R, 128), _NEG, jnp.float32)], axis=0
        )
    s0 = jnp.where(flat < N, p, _NEG)
    s_ref[...] = s0
    cm_ref[...] = jnp.max(s0.reshape(CH, 8, 128), axis=1)

    chunk_i = lax.broadcasted_iota(jnp.int32, (CH, 128), 0)
    rl8 = (
        lax.broadcasted_iota(jnp.int32, (8, 128), 0) * 128
        + lax.broadcasted_iota(jnp.int32, (8, 128), 1)
    )
    sub8 = lax.broadcasted_iota(jnp.int32, (8, 128), 0)
    li8 = lax.broadcasted_iota(jnp.int32, (8, 128), 1)
    li = lax.broadcasted_iota(jnp.int32, (1, 128), 1)
    BIG = jnp.int32(2**30)

    def body(k, carry):
        av, aidx = carry
        cm = cm_ref[...]
        m = jnp.max(cm)
        chunk = jnp.min(jnp.where(cm == m, chunk_i, BIG))
        sc = s_ref[pl.ds(chunk * 8, 8), :]
        rowlane = jnp.min(jnp.where(sc == m, rl8, BIG))
        r8 = rowlane // 128
        l = rowlane % 128
        idx = (chunk * 8 + r8) * 128 + l
        sc2 = jnp.where((sub8 == r8) & (li8 == l), _NEG, sc)
        s_ref[pl.ds(chunk * 8, 8), :] = sc2
        cm_ref[pl.ds(chunk, 1), :] = jnp.max(sc2, axis=0, keepdims=True)
        km = li == k
        av = jnp.where(km, m, av)
        aidx = jnp.where(km, idx, aidx)
        return (av, aidx)

    av, aidx = lax.fori_loop(
        0,
        K,
        body,
        (jnp.zeros((1, 128), jnp.float32), jnp.zeros((1, 128), jnp.int32)),
    )

    rowk = aidx // 128
    lanek = aidx % 128
    row_i160 = lax.broadcasted_iota(jnp.int32, (R, 128), 0)
    Or = (row_i160 == jnp.broadcast_to(rowk, (R, 128))).astype(jnp.float32)
    sub128 = lax.broadcasted_iota(jnp.int32, (128, 128), 0)
    Lm = (sub128 == jnp.broadcast_to(lanek, (128, 128))).astype(jnp.float32)
    ones128 = jnp.ones((128, 128), jnp.float32)

    def gather(coord):
        G = jnp.dot(
            boxT_ref[coord], Or,
            preferred_element_type=jnp.float32,
            precision=lax.Precision.HIGHEST,
        )
        masked = G * Lm
        row = jnp.sum(masked, axis=0, keepdims=True)
        colb = lax.dot_general(
            masked, ones128, (((0,), (0,)), ((), ())),
            preferred_element_type=jnp.float32,
            precision=lax.Precision.HIGHEST,
        )
        return row, colb

    x1r, x1c = gather(0)
    y1r, y1c = gather(1)
    x2r, x2c = gather(2)
    y2r, y2c = gather(3)

    area_r = jnp.clip(x2r - x1r, 0.0, None) * jnp.clip(y2r - y1r, 0.0, None)
    area_c = jnp.clip(x2c - x1c, 0.0, None) * jnp.clip(y2c - y1c, 0.0, None)

    w = jnp.clip(jnp.minimum(x2c, x2r) - jnp.maximum(x1c, x1r), 0.0, None)
    h = jnp.clip(jnp.minimum(y2c, y2r) - jnp.maximum(y1c, y1r), 0.0, None)
    inter = w * h
    iou_ref[...] = inter / (area_c + area_r - inter + 1e-8)

    def nbody(i, keep):
        row = iou_ref[pl.ds(i, 1), :]
        ki = jnp.sum(jnp.where(li == i, keep, 0.0))
        supp = (row > _IOU_T) & (ki > 0.0) & (li > i)
        return jnp.where(supp, 0.0, keep)

    keep = lax.fori_loop(0, K, nbody, jnp.ones((1, 128), jnp.float32))
    keepf = keep * (av > _SCORE_T).astype(jnp.float32)
    ax1, ay1, ax2, ay2 = x1r, y1r, x2r, y2r

    out = jnp.concatenate(
        [ax1 * keepf, ay1 * keepf, ax2 * keepf, ay2 * keepf, av * keepf,
         jnp.zeros((3, 128), jnp.float32)],
        axis=0,
    )
    out_ref[...] = out.reshape(1, 8, 128)


def kernel(cls_logits, bbox_pred, priors):
    N, C = cls_logits.shape[1], cls_logits.shape[2]
    R = -(-N // 128)
    Np = R * 128

    ltT = jnp.pad(cls_logits[0], ((0, Np - N), (0, 0))).T.reshape(C, R, 128)
    bbT = jnp.pad(bbox_pred[0], ((0, Np - N), (0, 0))).T.reshape(4, R, 128)
    prT = jnp.pad(priors, ((0, Np - N), (0, 0))).T.reshape(4, R, 128)

    RB = R
    for cand in (32, 16, 8, 4, 2):
        if R % cand == 0:
            RB = cand
            break

    probsT, boxes4 = pl.pallas_call(
        functools.partial(_prologue, C=C),
        grid=(R // RB,),
        in_specs=[
            pl.BlockSpec((C, RB, 128), lambda i: (0, i, 0)),
            pl.BlockSpec((4, RB, 128), lambda i: (0, i, 0)),
            pl.BlockSpec((4, RB, 128), lambda i: (0, i, 0)),
        ],
        out_specs=[
            pl.BlockSpec((C, RB, 128), lambda i: (0, i, 0)),
            pl.BlockSpec((4, RB, 128), lambda i: (0, i, 0)),
        ],
        out_shape=[
            jax.ShapeDtypeStruct((C, R, 128), jnp.float32),
            jax.ShapeDtypeStruct((4, R, 128), jnp.float32),
        ],
        compiler_params=pltpu.CompilerParams(
            dimension_semantics=("parallel",)
        ),
    )(ltT, bbT, prT)

    boxesT = jnp.transpose(boxes4, (0, 2, 1))

    CH = -(-R // 8)
    out = pl.pallas_call(
        functools.partial(_perclass, N=N, R=R, K=_TOPK),
        grid=(C - 1,),
        in_specs=[
            pl.BlockSpec((1, R, 128), lambda c: (c + 1, 0, 0)),
            pl.BlockSpec((4, 128, R), lambda c: (0, 0, 0)),
        ],
        out_specs=pl.BlockSpec((1, 8, 128), lambda c: (c, 0, 0)),
        out_shape=jax.ShapeDtypeStruct((C - 1, 8, 128), jnp.float32),
        scratch_shapes=[
            pltpu.VMEM((CH * 8, 128), jnp.float32),
            pltpu.VMEM((CH, 128), jnp.float32),
            pltpu.VMEM((128, 128), jnp.float32),
        ],
        compiler_params=pltpu.CompilerParams(
            dimension_semantics=("parallel",)
        ),
    )(probsT, boxesT)

    return out[:, :5, :_TOPK].transpose(0, 2, 1)


# speedup vs baseline: 1.4045x; 1.4045x over previous
import functools

import jax
import jax.numpy as jnp
from jax import lax
from jax.experimental import pallas as pl
from jax.experimental.pallas import tpu as pltpu

_CENTER_VAR = 0.1
_SIZE_VAR = 0.2
_IOU_T = 0.45
_SCORE_T = 0.01
_TOPK = 100
_NEG = -1e30


def _prologue(lt_ref, bb_ref, pr_ref, probs_ref, box_ref, *, C):
    x = lt_ref[...]
    m = jnp.max(x, axis=0, keepdims=True)
    e = jnp.exp(x - m)
    d = jnp.sum(e, axis=0, keepdims=True)
    probs_ref[...] = e / d

    loc = bb_ref[...]
    p = pr_ref[...]
    cx = loc[0] * _CENTER_VAR * p[2] + p[0]
    cy = loc[1] * _CENTER_VAR * p[3] + p[1]
    w = jnp.exp(loc[2] * _SIZE_VAR) * p[2]
    h = jnp.exp(loc[3] * _SIZE_VAR) * p[3]
    box_ref[...] = jnp.stack(
        [cx - w * 0.5, cy - h * 0.5, cx + w * 0.5, cy + h * 0.5], axis=0
    )


def _perclass(probs_ref, boxT_ref, out_ref, iou_ref, *, N, R, K):
    rows_i = lax.broadcasted_iota(jnp.int32, (R, 128), 0)
    lanes_i = lax.broadcasted_iota(jnp.int32, (R, 128), 1)
    flat = rows_i * 128 + lanes_i

    s0 = jnp.where(flat < N, probs_ref[0], _NEG)

    li = lax.broadcasted_iota(jnp.int32, (1, 128), 1)
    BIG = jnp.int32(2**30)

    def body(k, carry):
        s, av, aidx = carry
        m = jnp.max(s, axis=(0, 1), keepdims=True)
        idxv = jnp.min(
            jnp.where(s == m, flat, BIG), axis=(0, 1), keepdims=True
        )
        s = jnp.where(flat == idxv, _NEG, s)
        km = li == k
        av = jnp.where(km, m, av)
        aidx = jnp.where(km, idxv, aidx)
        return (s, av, aidx)

    _, av, aidx = lax.fori_loop(
        0,
        K,
        body,
        (s0, jnp.zeros((1, 128), jnp.float32), jnp.zeros((1, 128), jnp.int32)),
    )

    rowk = aidx // 128
    lanek = aidx % 128
    row_i160 = lax.broadcasted_iota(jnp.int32, (R, 128), 0)
    Or = (row_i160 == jnp.broadcast_to(rowk, (R, 128))).astype(jnp.float32)
    sub128 = lax.broadcasted_iota(jnp.int32, (128, 128), 0)
    Lm = (sub128 == jnp.broadcast_to(lanek, (128, 128))).astype(jnp.float32)
    ones128 = jnp.ones((128, 128), jnp.float32)

    def gather(coord):
        G = jnp.dot(
            boxT_ref[coord], Or,
            preferred_element_type=jnp.float32,
            precision=lax.Precision.HIGHEST,
        )
        masked = G * Lm
        row = jnp.sum(masked, axis=0, keepdims=True)
        colb = lax.dot_general(
            masked, ones128, (((0,), (0,)), ((), ())),
            preferred_element_type=jnp.float32,
            precision=lax.Precision.HIGHEST,
        )
        return row, colb

    x1r, x1c = gather(0)
    y1r, y1c = gather(1)
    x2r, x2c = gather(2)
    y2r, y2c = gather(3)

    area_r = jnp.clip(x2r - x1r, 0.0, None) * jnp.clip(y2r - y1r, 0.0, None)
    area_c = jnp.clip(x2c - x1c, 0.0, None) * jnp.clip(y2c - y1c, 0.0, None)

    w = jnp.clip(jnp.minimum(x2c, x2r) - jnp.maximum(x1c, x1r), 0.0, None)
    h = jnp.clip(jnp.minimum(y2c, y2r) - jnp.maximum(y1c, y1r), 0.0, None)
    inter = w * h
    iou_ref[...] = inter / (area_c + area_r - inter + 1e-8)

    def nbody(i, keep):
        row = iou_ref[pl.ds(i, 1), :]
        ki = jnp.sum(jnp.where(li == i, keep, 0.0), axis=(0, 1), keepdims=True)
        supp = (row > _IOU_T) & (ki > 0.0) & (li > i)
        return jnp.where(supp, 0.0, keep)

    keep = lax.fori_loop(0, K, nbody, jnp.ones((1, 128), jnp.float32))
    keepf = keep * (av > _SCORE_T).astype(jnp.float32)
    ax1, ay1, ax2, ay2 = x1r, y1r, x2r, y2r

    out = jnp.concatenate(
        [ax1 * keepf, ay1 * keepf, ax2 * keepf, ay2 * keepf, av * keepf,
         jnp.zeros((3, 128), jnp.float32)],
        axis=0,
    )
    out_ref[...] = out.reshape(1, 8, 128)


def kernel(cls_logits, bbox_pred, priors):
    N, C = cls_logits.shape[1], cls_logits.shape[2]
    R = -(-N // 128)
    Np = R * 128

    ltT = jnp.pad(cls_logits[0], ((0, Np - N), (0, 0))).T.reshape(C, R, 128)
    bbT = jnp.pad(bbox_pred[0], ((0, Np - N), (0, 0))).T.reshape(4, R, 128)
    prT = jnp.pad(priors, ((0, Np - N), (0, 0))).T.reshape(4, R, 128)

    RB = R
    for cand in (32, 16, 8, 4, 2):
        if R % cand == 0:
            RB = cand
            break

    probsT, boxes4 = pl.pallas_call(
        functools.partial(_prologue, C=C),
        grid=(R // RB,),
        in_specs=[
            pl.BlockSpec((C, RB, 128), lambda i: (0, i, 0)),
            pl.BlockSpec((4, RB, 128), lambda i: (0, i, 0)),
            pl.BlockSpec((4, RB, 128), lambda i: (0, i, 0)),
        ],
        out_specs=[
            pl.BlockSpec((C, RB, 128), lambda i: (0, i, 0)),
            pl.BlockSpec((4, RB, 128), lambda i: (0, i, 0)),
        ],
        out_shape=[
            jax.ShapeDtypeStruct((C, R, 128), jnp.float32),
            jax.ShapeDtypeStruct((4, R, 128), jnp.float32),
        ],
        compiler_params=pltpu.CompilerParams(
            dimension_semantics=("parallel",)
        ),
    )(ltT, bbT, prT)

    boxesT = jnp.transpose(boxes4, (0, 2, 1))

    out = pl.pallas_call(
        functools.partial(_perclass, N=N, R=R, K=_TOPK),
        grid=(C - 1,),
        in_specs=[
            pl.BlockSpec((1, R, 128), lambda c: (c + 1, 0, 0)),
            pl.BlockSpec((4, 128, R), lambda c: (0, 0, 0)),
        ],
        out_specs=pl.BlockSpec((1, 8, 128), lambda c: (c, 0, 0)),
        out_shape=jax.ShapeDtypeStruct((C - 1, 8, 128), jnp.float32),
        scratch_shapes=[
            pltpu.VMEM((128, 128), jnp.float32),
        ],
        compiler_params=pltpu.CompilerParams(
            dimension_semantics=("parallel",)
        ),
    )(probsT, boxesT)

    return out[:, :5, :_TOPK].transpose(0, 2, 1)
